# baseline (speedup 1.0000x reference)
import functools

import jax
import jax.numpy as jnp
from jax import lax
from jax.experimental import pallas as pl
from jax.experimental.pallas import tpu as pltpu
from jax.experimental.pallas import tpu_sc as plsc

N = 10000
E = 320000
D_IN = 128
D_OUT = 128
H = 32

NC, NS = 2, 16
NW = NC * NS
CH = 128
NCHUNK = 81
EPW = NCHUNK * CH
EP = NW * EPW
NP = 10240
RPS = NP // NS
N2 = N // 2
NP2 = NP // 2
RPS2 = RPS // 2


@functools.cache
def _get_sc_aggregate():
    mesh = plsc.VectorSubcoreMesh(
        core_axis_name="c", subcore_axis_name="s",
        num_cores=NC, num_subcores=NS)
    nbuf = 3

    @functools.partial(
        pl.kernel,
        out_type=jax.ShapeDtypeStruct((NC, NP, 2 * H), jnp.float32),
        mesh=mesh,
        scratch_types=[
            pltpu.VMEM((NCHUNK, CH), jnp.int32),
            pltpu.VMEM((NCHUNK, CH), jnp.int32),
            [pltpu.VMEM((CH, 2 * H), jnp.float32) for _ in range(nbuf)],
            pltpu.VMEM_SHARED((NP, 2 * H), jnp.float32),
            pltpu.VMEM_SHARED((NP, 2 * H), jnp.float32),
            [pltpu.SemaphoreType.DMA for _ in range(nbuf)],
            [pltpu.SemaphoreType.DMA for _ in range(nbuf)],
            pltpu.SemaphoreType.DMA,
        ],
        compiler_params=pltpu.CompilerParams(use_tc_tiling_on_sc=False),
    )
    def _sc_aggregate(u_hbm, pk_hbm, out_hbm,
                      src_v, dst_v, rows, acc_sh, u_sh, gsem, ssem, usem):
        c = lax.axis_index("c")
        s = lax.axis_index("s")
        wid = c * NS + s
        u_stage = pltpu.async_copy(u_hbm.at[pl.ds(s * RPS, RPS)],
                                   u_sh.at[pl.ds(s * RPS, RPS)], usem)
        pk_load = pltpu.async_copy(pk_hbm.at[wid], src_v, gsem[0])

        def zrow(j, carry):
            for k in range(4):
                rows[0][j, pl.ds(k * 16, 16)] = jnp.zeros((16,), jnp.float32)
            return carry

        lax.fori_loop(0, CH, zrow, 0)
        zc = [pltpu.async_copy(rows[0], acc_sh.at[pl.ds(s * RPS + t * CH, CH)],
                               ssem[0]) for t in range(RPS // CH)]

        pk_load.wait()

        def unpack(j, carry):
            for k in range(CH // 16):
                p = src_v[j, pl.ds(k * 16, 16)]
                src_v[j, pl.ds(k * 16, 16)] = p & 0xFFFF
                dst_v[j, pl.ds(k * 16, 16)] = p >> 16
            return carry

        lax.fori_loop(0, NCHUNK, unpack, 0)
        for d in zc:
            d.wait()
        u_stage.wait()
        plsc.subcore_barrier()

        def g_start(j, b):
            pltpu.async_copy(u_sh.at[src_v.at[j]], rows[b], gsem[b])

        def g_wait(j, b):
            pltpu.make_async_copy(u_sh.at[src_v.at[j]], rows[b], gsem[b]).wait()

        def s_start(j, b):
            pltpu.async_copy(rows[b], acc_sh.at[dst_v.at[j]], ssem[b], add=True)

        def s_wait(j, b):
            pltpu.make_async_copy(rows[b], acc_sh.at[dst_v.at[j]], ssem[b]).wait()

        for k in range(nbuf):
            g_start(k, k)
        for k in range(nbuf):
            g_wait(k, k)
            s_start(k, k)

        def group_body(i, carry):
            j0 = nbuf * i
            for k in range(nbuf):
                s_wait(j0 + k, k)
                g_start(j0 + k, k)
            for k in range(nbuf):
                g_wait(j0 + k, k)
                s_start(j0 + k, k)
            return carry

        lax.fori_loop(1, NCHUNK // nbuf, group_body, 0)
        for k in range(nbuf):
            s_wait(NCHUNK - nbuf + k, k)
        plsc.subcore_barrier()
        pltpu.sync_copy(acc_sh.at[pl.ds(s * RPS, RPS)],
                        out_hbm.at[c, pl.ds(s * RPS, RPS)])

    return _sc_aggregate


def _build_u(xp, u_ref):
    r = jnp.maximum(xp, 0.0) + 1e-7
    g = jnp.max(r)
    q = jnp.exp(r - g)
    p = q * r
    u2 = jnp.concatenate(
        [q[:, 0:H], p[:, 0:H], q[:, H:2 * H], p[:, H:2 * H]], axis=1)
    u_ref[0:N2, :] = u2
    u_ref[N2:NP2, :] = jnp.zeros((NP2 - N2, 4 * H), jnp.float32)


def _pre_body(xv_ref, w0_ref, b0_ref, xp_ref, u_ref):
    xp = jnp.dot(xv_ref[...], w0_ref[...],
                 preferred_element_type=jnp.float32) + b0_ref[...]
    xp_ref[...] = xp
    _build_u(xp, u_ref)


_pre_call = pl.pallas_call(
    _pre_body,
    out_shape=(jax.ShapeDtypeStruct((N2, 2 * H), jnp.float32),
               jax.ShapeDtypeStruct((NP2, 4 * H), jnp.float32)),
)


def _conv_dense(part, xp, w1d, b1d, gd, bed, w2d, b2d):
    sall = part[0, 0:N2, :] + part[1, 0:N2, :]
    den = jnp.concatenate([sall[:, 0:H], sall[:, 2 * H:3 * H]], axis=1)
    num = jnp.concatenate([sall[:, H:2 * H], sall[:, 3 * H:4 * H]], axis=1)
    aggr = num / (den + 1e-30)
    h = aggr + xp
    h1 = jnp.dot(h, w1d, preferred_element_type=jnp.float32) + b1d
    mean = jnp.mean(h1, axis=0, keepdims=True)
    meanb = (mean[:, 0:2 * H] + mean[:, 2 * H:4 * H]) * 0.5
    meanb = jnp.concatenate([meanb, meanb], axis=1)
    d = h1 - meanb
    var = jnp.mean(d * d, axis=0, keepdims=True)
    varb = (var[:, 0:2 * H] + var[:, 2 * H:4 * H]) * 0.5
    varb = jnp.concatenate([varb, varb], axis=1)
    hn = d * (gd / jnp.sqrt(varb + 1e-5)) + bed
    hr = jnp.maximum(hn, 0.0)
    h2 = jnp.dot(hr, w2d, preferred_element_type=jnp.float32) + b2d
    return jnp.maximum(h2, 0.0)


def _mid_body(part_ref, xp_ref, w1_ref, b1_ref, g_ref, be_ref, w2_ref, b2_ref,
              xn_ref, u_ref):
    xn = _conv_dense(part_ref[...], xp_ref[...], w1_ref[...], b1_ref[...],
                     g_ref[...], be_ref[...], w2_ref[...], b2_ref[...])
    xn_ref[...] = xn
    _build_u(xn, u_ref)


_mid_call = pl.pallas_call(
    _mid_body,
    out_shape=(jax.ShapeDtypeStruct((N2, 2 * H), jnp.float32),
               jax.ShapeDtypeStruct((NP2, 4 * H), jnp.float32)),
)


def _last_body(part_ref, xp_ref, w1_ref, b1_ref, g_ref, be_ref, w2_ref, b2_ref,
               w4_ref, b4_ref, out_ref):
    x3 = _conv_dense(part_ref[...], xp_ref[...], w1_ref[...], b1_ref[...],
                     g_ref[...], be_ref[...], w2_ref[...], b2_ref[...])
    out_ref[...] = jnp.dot(x3, w4_ref[...],
                           preferred_element_type=jnp.float32) + b4_ref[...]


_last_call = pl.pallas_call(
    _last_body,
    out_shape=jax.ShapeDtypeStruct((N2, 2 * D_OUT), jnp.float32),
)


def _dbl_w(w):
    a, b = w.shape
    z = jnp.zeros((a, b), jnp.float32)
    return jnp.concatenate(
        [jnp.concatenate([w, z], axis=1), jnp.concatenate([z, w], axis=1)],
        axis=0)


def _dbl_b(b):
    return jnp.concatenate([b, b]).reshape(1, -1)


def kernel(x, edge_index, w0, b0,
           c1_w1, c1_b1, c1_g, c1_be, c1_w2, c1_b2,
           c2_w1, c2_b1, c2_g, c2_be, c2_w2, c2_b2,
           c3_w1, c3_b1, c3_g, c3_be, c3_w2, c3_b2,
           w4, b4):
    pk = edge_index[0] | (edge_index[1] << 16)
    padrow = N + jnp.arange(EP - E, dtype=jnp.int32) % (NP - N)
    pkr = jnp.concatenate([pk, padrow | (padrow << 16)]).reshape(NW, NCHUNK, CH)

    xp, u = _pre_call(jnp.reshape(x, (N2, 2 * D_IN)), _dbl_w(w0), _dbl_b(b0))
    convs = [
        (c1_w1, c1_b1, c1_g, c1_be, c1_w2, c1_b2),
        (c2_w1, c2_b1, c2_g, c2_be, c2_w2, c2_b2),
        (c3_w1, c3_b1, c3_g, c3_be, c3_w2, c3_b2),
    ]
    sc_aggregate = _get_sc_aggregate()
    for layer, (w1, b1, g, be, w2, b2) in enumerate(convs):
        part = sc_aggregate(jnp.reshape(u, (NP, 2 * H)), pkr)
        part = jnp.reshape(part, (NC, NP2, 4 * H))
        args = (part, xp, _dbl_w(w1), _dbl_b(b1), _dbl_b(g), _dbl_b(be),
                _dbl_w(w2), _dbl_b(b2))
        if layer < 2:
            xp, u = _mid_call(*args)
        else:
            out = _last_call(*args, _dbl_w(w4), _dbl_b(b4))
    return jnp.reshape(out, (N, D_OUT))


# speedup vs baseline: 1.1336x; 1.1336x over previous
import functools

import jax
import jax.numpy as jnp
from jax import lax
from jax.experimental import pallas as pl
from jax.experimental.pallas import tpu as pltpu
from jax.experimental.pallas import tpu_sc as plsc

N = 10000
E = 320000
D_IN = 128
D_OUT = 128
H = 32

NC, NS = 2, 16
NW = NC * NS
CH = 128
NCHUNK = 81
EPW = NCHUNK * CH
EP = NW * EPW
NP = 10240
RPS = NP // NS
N2 = N // 2
NP2 = NP // 2
RPS2 = RPS // 2


@functools.cache
def _get_sc_aggregate():
    mesh = plsc.VectorSubcoreMesh(
        core_axis_name="c", subcore_axis_name="s",
        num_cores=NC, num_subcores=NS)
    @functools.partial(
        pl.kernel,
        out_type=jax.ShapeDtypeStruct((NC, NP, 2 * H), jnp.float32),
        mesh=mesh,
        scratch_types=[
            pltpu.VMEM((NCHUNK, CH), jnp.int32),
            pltpu.VMEM((NCHUNK, CH), jnp.int32),
            [pltpu.VMEM((CH, 2 * H), jnp.float32) for _ in range(2)],
            pltpu.VMEM_SHARED((NP, 2 * H), jnp.float32),
            pltpu.VMEM_SHARED((NP, 2 * H), jnp.float32),
            [pltpu.SemaphoreType.DMA for _ in range(2)],
            [pltpu.SemaphoreType.DMA for _ in range(2)],
            pltpu.SemaphoreType.DMA,
        ],
        compiler_params=pltpu.CompilerParams(use_tc_tiling_on_sc=False),
    )
    def _sc_aggregate(u_hbm, src_hbm, dst_hbm, z_hbm, out_hbm,
                      src_v, dst_v, rows, acc_sh, u_sh, gsem, ssem, usem):
        c = lax.axis_index("c")
        s = lax.axis_index("s")
        wid = c * NS + s
        u_stage = pltpu.async_copy(u_hbm.at[pl.ds(s * RPS, RPS)],
                                   u_sh.at[pl.ds(s * RPS, RPS)], usem)
        zc = pltpu.async_copy(z_hbm.at[pl.ds(s * RPS, RPS)],
                              acc_sh.at[pl.ds(s * RPS, RPS)], ssem[0])
        s_load = pltpu.async_copy(src_hbm.at[wid], src_v, gsem[0])
        d_load = pltpu.async_copy(dst_hbm.at[wid], dst_v, gsem[1])
        s_load.wait()
        d_load.wait()
        zc.wait()
        u_stage.wait()
        plsc.subcore_barrier()

        def g_start(j, b):
            pltpu.async_copy(u_sh.at[src_v.at[j]], rows[b], gsem[b])

        def g_wait(j, b):
            pltpu.make_async_copy(u_sh.at[src_v.at[j]], rows[b], gsem[b]).wait()

        def s_start(j, b):
            pltpu.async_copy(rows[b], acc_sh.at[dst_v.at[j]], ssem[b], add=True)

        def s_wait(j, b):
            pltpu.make_async_copy(rows[b], acc_sh.at[dst_v.at[j]], ssem[b]).wait()

        g_start(0, 0)
        g_start(1, 1)
        g_wait(0, 0)
        s_start(0, 0)
        g_wait(1, 1)
        s_start(1, 1)

        def pair_body(i, carry):
            j0 = 2 * i
            j1 = j0 + 1
            s_wait(j0, 0)
            g_start(j0, 0)
            s_wait(j1, 1)
            g_start(j1, 1)
            g_wait(j0, 0)
            s_start(j0, 0)
            g_wait(j1, 1)
            s_start(j1, 1)
            return carry

        lax.fori_loop(1, (NCHUNK - 1) // 2, pair_body, 0)
        s_wait(NCHUNK - 3, 0)
        g_start(NCHUNK - 1, 0)
        g_wait(NCHUNK - 1, 0)
        s_start(NCHUNK - 1, 0)
        s_wait(NCHUNK - 1, 0)
        s_wait(NCHUNK - 2, 1)
        plsc.subcore_barrier()
        pltpu.sync_copy(acc_sh.at[pl.ds(s * RPS, RPS)],
                        out_hbm.at[c, pl.ds(s * RPS, RPS)])

    return _sc_aggregate


def _build_u(xp, u_ref):
    r = jnp.maximum(xp, 0.0) + 1e-7
    g = jnp.max(r)
    q = jnp.exp(r - g)
    p = q * r
    u2 = jnp.concatenate(
        [q[:, 0:H], p[:, 0:H], q[:, H:2 * H], p[:, H:2 * H]], axis=1)
    u_ref[0:N2, :] = u2
    u_ref[N2:NP2, :] = jnp.zeros((NP2 - N2, 4 * H), jnp.float32)


def _pre_body(xv_ref, w0_ref, b0_ref, xp_ref, u_ref):
    xp = jnp.dot(xv_ref[...], w0_ref[...],
                 preferred_element_type=jnp.float32) + b0_ref[...]
    xp_ref[...] = xp
    _build_u(xp, u_ref)


_pre_call = pl.pallas_call(
    _pre_body,
    out_shape=(jax.ShapeDtypeStruct((N2, 2 * H), jnp.float32),
               jax.ShapeDtypeStruct((NP2, 4 * H), jnp.float32)),
)


def _conv_dense(part, xp, w1d, b1d, gd, bed, w2d, b2d):
    sall = part[0, 0:N2, :] + part[1, 0:N2, :]
    den = jnp.concatenate([sall[:, 0:H], sall[:, 2 * H:3 * H]], axis=1)
    num = jnp.concatenate([sall[:, H:2 * H], sall[:, 3 * H:4 * H]], axis=1)
    aggr = num / (den + 1e-30)
    h = aggr + xp
    h1 = jnp.dot(h, w1d, preferred_element_type=jnp.float32) + b1d
    mean = jnp.mean(h1, axis=0, keepdims=True)
    meanb = (mean[:, 0:2 * H] + mean[:, 2 * H:4 * H]) * 0.5
    meanb = jnp.concatenate([meanb, meanb], axis=1)
    d = h1 - meanb
    var = jnp.mean(d * d, axis=0, keepdims=True)
    varb = (var[:, 0:2 * H] + var[:, 2 * H:4 * H]) * 0.5
    varb = jnp.concatenate([varb, varb], axis=1)
    hn = d * (gd / jnp.sqrt(varb + 1e-5)) + bed
    hr = jnp.maximum(hn, 0.0)
    h2 = jnp.dot(hr, w2d, preferred_element_type=jnp.float32) + b2d
    return jnp.maximum(h2, 0.0)


def _mid_body(part_ref, xp_ref, w1_ref, b1_ref, g_ref, be_ref, w2_ref, b2_ref,
              xn_ref, u_ref):
    xn = _conv_dense(part_ref[...], xp_ref[...], w1_ref[...], b1_ref[...],
                     g_ref[...], be_ref[...], w2_ref[...], b2_ref[...])
    xn_ref[...] = xn
    _build_u(xn, u_ref)


_mid_call = pl.pallas_call(
    _mid_body,
    out_shape=(jax.ShapeDtypeStruct((N2, 2 * H), jnp.float32),
               jax.ShapeDtypeStruct((NP2, 4 * H), jnp.float32)),
)


def _last_body(part_ref, xp_ref, w1_ref, b1_ref, g_ref, be_ref, w2_ref, b2_ref,
               w4_ref, b4_ref, out_ref):
    x3 = _conv_dense(part_ref[...], xp_ref[...], w1_ref[...], b1_ref[...],
                     g_ref[...], be_ref[...], w2_ref[...], b2_ref[...])
    out_ref[...] = jnp.dot(x3, w4_ref[...],
                           preferred_element_type=jnp.float32) + b4_ref[...]


_last_call = pl.pallas_call(
    _last_body,
    out_shape=jax.ShapeDtypeStruct((N2, 2 * D_OUT), jnp.float32),
)


def _dbl_w(w):
    a, b = w.shape
    z = jnp.zeros((a, b), jnp.float32)
    return jnp.concatenate(
        [jnp.concatenate([w, z], axis=1), jnp.concatenate([z, w], axis=1)],
        axis=0)


def _dbl_b(b):
    return jnp.concatenate([b, b]).reshape(1, -1)


def kernel(x, edge_index, w0, b0,
           c1_w1, c1_b1, c1_g, c1_be, c1_w2, c1_b2,
           c2_w1, c2_b1, c2_g, c2_be, c2_w2, c2_b2,
           c3_w1, c3_b1, c3_g, c3_be, c3_w2, c3_b2,
           w4, b4):
    padrow = N + jnp.arange(EP - E, dtype=jnp.int32) % (NP - N)
    srcr = jnp.concatenate([edge_index[0], padrow]).reshape(NW, NCHUNK, CH)
    dstr = jnp.concatenate([edge_index[1], padrow]).reshape(NW, NCHUNK, CH)
    z = jnp.zeros((NP, 2 * H), jnp.float32)

    xp, u = _pre_call(jnp.reshape(x, (N2, 2 * D_IN)), _dbl_w(w0), _dbl_b(b0))
    convs = [
        (c1_w1, c1_b1, c1_g, c1_be, c1_w2, c1_b2),
        (c2_w1, c2_b1, c2_g, c2_be, c2_w2, c2_b2),
        (c3_w1, c3_b1, c3_g, c3_be, c3_w2, c3_b2),
    ]
    sc_aggregate = _get_sc_aggregate()
    for layer, (w1, b1, g, be, w2, b2) in enumerate(convs):
        part = sc_aggregate(jnp.reshape(u, (NP, 2 * H)), srcr, dstr, z)
        part = jnp.reshape(part, (NC, NP2, 4 * H))
        args = (part, xp, _dbl_w(w1), _dbl_b(b1), _dbl_b(g), _dbl_b(be),
                _dbl_w(w2), _dbl_b(b2))
        if layer < 2:
            xp, u = _mid_call(*args)
        else:
            out = _last_call(*args, _dbl_w(w4), _dbl_b(b4))
    return jnp.reshape(out, (N, D_OUT))
